# Initial kernel scaffold; baseline (speedup 1.0000x reference)
import jax
import jax.numpy as jnp
from jax.experimental import pallas as pl


def kernel(hidden_states, top_k_index, top_k_weights, gate_up_proj, down_proj):
    raise NotImplementedError("write your pallas kernel here")


# speedup vs baseline: 1.2098x; 1.2098x over previous
import jax
import jax.numpy as jnp
from jax.experimental import pallas as pl

NUM_EXPERTS = 16
HIDDEN = 2048
INTER = 1024
TOP_K = 2
TOKENS = 8192
TM = 512


def _ffn_body(idx_ref, w_ref, x_ref, gu_ref, dp_ref, out_ref):
    e = pl.program_id(1)
    mask = (idx_ref[...] == e).astype(jnp.float32)
    w = jnp.sum(w_ref[...] * mask, axis=1, keepdims=True)
    x = x_ref[...].astype(jnp.bfloat16)
    gu = jnp.dot(x, gu_ref[0], preferred_element_type=jnp.float32)
    gate = gu[:, :INTER]
    up = gu[:, INTER:]
    h = (jax.nn.silu(gate) * up).astype(jnp.bfloat16)
    d = jnp.dot(h, dp_ref[0], preferred_element_type=jnp.float32)
    d = d * w

    @pl.when(e == 0)
    def _():
        out_ref[...] = d

    @pl.when(e > 0)
    def _():
        out_ref[...] += d


def kernel(hidden_states, top_k_index, top_k_weights, gate_up_proj, down_proj):
    gu = gate_up_proj.astype(jnp.bfloat16)
    dp = down_proj.astype(jnp.bfloat16)
    idx = top_k_index.astype(jnp.int32)
    grid = (TOKENS // TM, NUM_EXPERTS)
    return pl.pallas_call(
        _ffn_body,
        grid=grid,
        in_specs=[
            pl.BlockSpec((TM, TOP_K), lambda m, e: (m, 0)),
            pl.BlockSpec((TM, TOP_K), lambda m, e: (m, 0)),
            pl.BlockSpec((TM, HIDDEN), lambda m, e: (m, 0)),
            pl.BlockSpec((1, HIDDEN, 2 * INTER), lambda m, e: (e, 0, 0)),
            pl.BlockSpec((1, INTER, HIDDEN), lambda m, e: (e, 0, 0)),
        ],
        out_specs=pl.BlockSpec((TM, HIDDEN), lambda m, e: (m, 0)),
        out_shape=jax.ShapeDtypeStruct((TOKENS, HIDDEN), jnp.float32),
    )(idx, top_k_weights, hidden_states, gu, dp)
